# baseline (speedup 1.0000x reference)
import functools

import jax
import jax.numpy as jnp
import numpy as np
from jax import lax
from jax.experimental import pallas as pl
from jax.experimental.pallas import tpu as pltpu
from jax.experimental.pallas import tpu_sc as plsc

L = 65536
Q = 8388608
NC = 2
NS = 16
NW = NC * NS
QPW = Q // NW
CHUNK = 8192
NCHUNKS = QPW // CHUNK

H = np.float32(1.0 / (L - 1))
K = np.float32(L - 1)

_mesh = plsc.VectorSubcoreMesh(core_axis_name="c", subcore_axis_name="s")


@functools.partial(
    pl.kernel,
    out_type=jax.ShapeDtypeStruct((Q,), jnp.float32),
    mesh=_mesh,
    scratch_types=[
        pltpu.VMEM((L,), jnp.float32),
        pltpu.VMEM((CHUNK,), jnp.float32),
        pltpu.VMEM((CHUNK,), jnp.float32),
        pltpu.VMEM((CHUNK,), jnp.float32),
        pltpu.VMEM((CHUNK,), jnp.float32),
        pltpu.VMEM_SHARED((L,), jnp.float32),
        pltpu.SemaphoreType.DMA,
        pltpu.SemaphoreType.DMA,
        pltpu.SemaphoreType.DMA,
        pltpu.SemaphoreType.DMA,
    ],
    compiler_params=pltpu.CompilerParams(needs_layout_passes=False),
)
def _interp_sc(t_hbm, us_hbm, out_hbm, us_v, t0_v, t1_v, o0_v, o1_v,
               us_sh, si0, si1, so0, so1):
    wid = lax.axis_index("s") * NC + lax.axis_index("c")
    base = wid * QPW

    def compute(t_ref, o_ref):
        @plsc.parallel_loop(0, CHUNK, 16, unroll=16)
        def _(i):
            tq = t_ref[pl.ds(i, 16)]
            c = tq * K
            i0 = c.astype(jnp.int32)
            idx = i0 + 1
            yi = plsc.load_gather(us_v, [idx])
            ym = plsc.load_gather(us_v, [i0])
            dy = yi - ym
            off = c - idx.astype(jnp.float32)
            o_ref[pl.ds(i, 16)] = yi + dy * off

    def fetch(g, t_ref, sem):
        pltpu.async_copy(t_hbm.at[pl.ds(base + g * CHUNK, CHUNK)], t_ref, sem)

    def wait_in(t_ref, sem):
        pltpu.make_async_copy(t_hbm.at[pl.ds(0, CHUNK)], t_ref, sem).wait()

    def put(g, o_ref, sem):
        pltpu.async_copy(o_ref, out_hbm.at[pl.ds(base + g * CHUNK, CHUNK)], sem)

    def wait_out(o_ref, sem):
        pltpu.make_async_copy(o_ref, out_hbm.at[pl.ds(0, CHUNK)], sem).wait()

    sid = lax.axis_index("s")
    fetch(0, t0_v, si0)
    fetch(1, t1_v, si1)

    @pl.when(sid == 0)
    def _():
        pltpu.sync_copy(us_hbm, us_sh)

    plsc.subcore_barrier()
    pltpu.sync_copy(us_sh, us_v)
    lane = jnp.arange(16, dtype=jnp.int32)
    us_v[pl.ds(0, 16)] = plsc.load_gather(us_v, [jnp.maximum(lane, 1)])
    wait_in(t0_v, si0)
    compute(t0_v, o0_v)
    put(0, o0_v, so0)
    fetch(2, t0_v, si0)
    wait_in(t1_v, si1)
    compute(t1_v, o1_v)
    put(1, o1_v, so1)
    fetch(3, t1_v, si1)

    def body(k, carry):
        g0 = 2 * k
        wait_in(t0_v, si0)
        wait_out(o0_v, so0)
        compute(t0_v, o0_v)
        put(g0, o0_v, so0)

        @pl.when(g0 + 2 < NCHUNKS)
        def _():
            fetch(g0 + 2, t0_v, si0)

        wait_in(t1_v, si1)
        wait_out(o1_v, so1)
        compute(t1_v, o1_v)
        put(g0 + 1, o1_v, so1)

        @pl.when(g0 + 3 < NCHUNKS)
        def _():
            fetch(g0 + 3, t1_v, si1)

        return carry

    lax.fori_loop(1, NCHUNKS // 2, body, 0)
    wait_out(o0_v, so0)
    wait_out(o1_v, so1)


def kernel(x, t, us, t_range):
    return _interp_sc(t, us)


# speedup vs baseline: 1.5064x; 1.5064x over previous
import functools

import jax
import jax.numpy as jnp
import numpy as np
from jax import lax
from jax.experimental import pallas as pl
from jax.experimental.pallas import tpu as pltpu
from jax.experimental.pallas import tpu_sc as plsc

L = 65536
Q = 8388608
NC = 2
NS = 16
NW = NC * NS
QPW = Q // NW
CHUNK = 4096
NCHUNKS = QPW // CHUNK

H = np.float32(1.0 / (L - 1))
K = np.float32(L - 1)

_mesh = plsc.VectorSubcoreMesh(core_axis_name="c", subcore_axis_name="s")


@functools.partial(
    pl.kernel,
    out_type=jax.ShapeDtypeStruct((Q,), jnp.float32),
    mesh=_mesh,
    scratch_types=[
        pltpu.VMEM((L,), jnp.float32),
        pltpu.VMEM((CHUNK,), jnp.float32),
        pltpu.VMEM((CHUNK,), jnp.float32),
        pltpu.VMEM((CHUNK,), jnp.float32),
        pltpu.VMEM((CHUNK,), jnp.float32),
        pltpu.VMEM_SHARED((L,), jnp.float32),
        pltpu.SemaphoreType.DMA,
        pltpu.SemaphoreType.DMA,
        pltpu.SemaphoreType.DMA,
        pltpu.SemaphoreType.DMA,
    ],
    compiler_params=pltpu.CompilerParams(needs_layout_passes=False),
)
def _interp_sc(t_hbm, us_hbm, out_hbm, us_v, t0_v, t1_v, o0_v, o1_v,
               us_sh, si0, si1, so0, so1):
    wid = lax.axis_index("s") * NC + lax.axis_index("c")
    base = wid * QPW

    def compute(t_ref, o_ref):
        @plsc.parallel_loop(0, CHUNK, 16, unroll=12)
        def _(i):
            tq = t_ref[pl.ds(i, 16)]
            c = tq * K
            i0 = c.astype(jnp.int32)
            idx = i0 + 1
            yi = plsc.load_gather(us_v, [idx])
            ym = plsc.load_gather(us_v, [i0])
            dy = yi - ym
            off = c - idx.astype(jnp.float32)
            o_ref[pl.ds(i, 16)] = yi + dy * off

    def fetch(g, t_ref, sem):
        pltpu.async_copy(t_hbm.at[pl.ds(base + g * CHUNK, CHUNK)], t_ref, sem)

    def wait_in(t_ref, sem):
        pltpu.make_async_copy(t_hbm.at[pl.ds(0, CHUNK)], t_ref, sem).wait()

    def put(g, o_ref, sem):
        pltpu.async_copy(o_ref, out_hbm.at[pl.ds(base + g * CHUNK, CHUNK)], sem)

    def wait_out(o_ref, sem):
        pltpu.make_async_copy(o_ref, out_hbm.at[pl.ds(0, CHUNK)], sem).wait()

    sid = lax.axis_index("s")
    fetch(0, t0_v, si0)
    fetch(1, t1_v, si1)

    @pl.when(sid == 0)
    def _():
        pltpu.sync_copy(us_hbm, us_sh)

    plsc.subcore_barrier()
    pltpu.sync_copy(us_sh, us_v)
    lane = jnp.arange(16, dtype=jnp.int32)
    us_v[pl.ds(0, 16)] = plsc.load_gather(us_v, [jnp.maximum(lane, 1)])
    wait_in(t0_v, si0)
    compute(t0_v, o0_v)
    put(0, o0_v, so0)
    fetch(2, t0_v, si0)
    wait_in(t1_v, si1)
    compute(t1_v, o1_v)
    put(1, o1_v, so1)
    fetch(3, t1_v, si1)

    def body(k, carry):
        g0 = 2 * k
        wait_in(t0_v, si0)
        wait_out(o0_v, so0)
        compute(t0_v, o0_v)
        put(g0, o0_v, so0)

        @pl.when(g0 + 2 < NCHUNKS)
        def _():
            fetch(g0 + 2, t0_v, si0)

        wait_in(t1_v, si1)
        wait_out(o1_v, so1)
        compute(t1_v, o1_v)
        put(g0 + 1, o1_v, so1)

        @pl.when(g0 + 3 < NCHUNKS)
        def _():
            fetch(g0 + 3, t1_v, si1)

        return carry

    lax.fori_loop(1, NCHUNKS // 2, body, 0)
    wait_out(o0_v, so0)
    wait_out(o1_v, so1)


def kernel(x, t, us, t_range):
    return _interp_sc(t, us)


# speedup vs baseline: 1.7561x; 1.1658x over previous
import functools

import jax
import jax.numpy as jnp
import numpy as np
from jax import lax
from jax.experimental import pallas as pl
from jax.experimental.pallas import tpu as pltpu
from jax.experimental.pallas import tpu_sc as plsc

L = 65536
Q = 8388608
NC = 2
NS = 16
NW = NC * NS
QPW = Q // NW
CHUNK = 8192
NCHUNKS = QPW // CHUNK

H = np.float32(1.0 / (L - 1))
K = np.float32(L - 1)

_mesh = plsc.VectorSubcoreMesh(core_axis_name="c", subcore_axis_name="s")


@functools.partial(
    pl.kernel,
    out_type=jax.ShapeDtypeStruct((Q,), jnp.float32),
    mesh=_mesh,
    scratch_types=[
        pltpu.VMEM((L,), jnp.float32),
        pltpu.VMEM((CHUNK,), jnp.float32),
        pltpu.VMEM((CHUNK,), jnp.float32),
        pltpu.VMEM((CHUNK,), jnp.float32),
        pltpu.VMEM((CHUNK,), jnp.float32),
        pltpu.VMEM_SHARED((L,), jnp.float32),
        pltpu.SemaphoreType.DMA,
        pltpu.SemaphoreType.DMA,
        pltpu.SemaphoreType.DMA,
        pltpu.SemaphoreType.DMA,
    ],
    compiler_params=pltpu.CompilerParams(needs_layout_passes=False),
)
def _interp_sc(t_hbm, us_hbm, out_hbm, us_v, t0_v, t1_v, o0_v, o1_v,
               us_sh, si0, si1, so0, so1):
    wid = lax.axis_index("s") * NC + lax.axis_index("c")
    base = wid * QPW

    def compute(t_ref, o_ref):
        @plsc.parallel_loop(0, CHUNK, 16, unroll=12)
        def _(i):
            tq = t_ref[pl.ds(i, 16)]
            c = tq * K
            i0 = c.astype(jnp.int32)
            idx = i0 + 1
            yi = plsc.load_gather(us_v, [idx])
            ym = plsc.load_gather(us_v, [i0])
            dy = yi - ym
            off = c - idx.astype(jnp.float32)
            o_ref[pl.ds(i, 16)] = yi + dy * off

    def fetch(g, t_ref, sem):
        pltpu.async_copy(t_hbm.at[pl.ds(base + g * CHUNK, CHUNK)], t_ref, sem)

    def wait_in(t_ref, sem):
        pltpu.make_async_copy(t_hbm.at[pl.ds(0, CHUNK)], t_ref, sem).wait()

    def put(g, o_ref, sem):
        pltpu.async_copy(o_ref, out_hbm.at[pl.ds(base + g * CHUNK, CHUNK)], sem)

    def wait_out(o_ref, sem):
        pltpu.make_async_copy(o_ref, out_hbm.at[pl.ds(0, CHUNK)], sem).wait()

    sid = lax.axis_index("s")
    fetch(0, t0_v, si0)
    fetch(1, t1_v, si1)

    @pl.when(sid == 0)
    def _():
        pltpu.sync_copy(us_hbm, us_sh)

    plsc.subcore_barrier()
    pltpu.sync_copy(us_sh, us_v)
    lane = jnp.arange(16, dtype=jnp.int32)
    us_v[pl.ds(0, 16)] = plsc.load_gather(us_v, [jnp.maximum(lane, 1)])
    wait_in(t0_v, si0)
    compute(t0_v, o0_v)
    put(0, o0_v, so0)
    fetch(2, t0_v, si0)
    wait_in(t1_v, si1)
    compute(t1_v, o1_v)
    put(1, o1_v, so1)
    fetch(3, t1_v, si1)

    def body(k, carry):
        g0 = 2 * k
        wait_in(t0_v, si0)
        wait_out(o0_v, so0)
        compute(t0_v, o0_v)
        put(g0, o0_v, so0)

        @pl.when(g0 + 2 < NCHUNKS)
        def _():
            fetch(g0 + 2, t0_v, si0)

        wait_in(t1_v, si1)
        wait_out(o1_v, so1)
        compute(t1_v, o1_v)
        put(g0 + 1, o1_v, so1)

        @pl.when(g0 + 3 < NCHUNKS)
        def _():
            fetch(g0 + 3, t1_v, si1)

        return carry

    lax.fori_loop(1, NCHUNKS // 2, body, 0)
    wait_out(o0_v, so0)
    wait_out(o1_v, so1)


def kernel(x, t, us, t_range):
    return _interp_sc(t, us)


# speedup vs baseline: 1.7575x; 1.0008x over previous
import functools

import jax
import jax.numpy as jnp
import numpy as np
from jax import lax
from jax.experimental import pallas as pl
from jax.experimental.pallas import tpu as pltpu
from jax.experimental.pallas import tpu_sc as plsc

L = 65536
Q = 8388608
NC = 2
NS = 16
NW = NC * NS
QPW = Q // NW
CHUNK = 8192
NCHUNKS = QPW // CHUNK

H = np.float32(1.0 / (L - 1))
K = np.float32(L - 1)

_mesh = plsc.VectorSubcoreMesh(core_axis_name="c", subcore_axis_name="s")


@functools.partial(
    pl.kernel,
    out_type=jax.ShapeDtypeStruct((Q,), jnp.float32),
    mesh=_mesh,
    scratch_types=[
        pltpu.VMEM((L,), jnp.float32),
        pltpu.VMEM((CHUNK,), jnp.float32),
        pltpu.VMEM((CHUNK,), jnp.float32),
        pltpu.VMEM((CHUNK,), jnp.float32),
        pltpu.VMEM((CHUNK,), jnp.float32),
        pltpu.VMEM_SHARED((L,), jnp.float32),
        pltpu.SemaphoreType.DMA,
        pltpu.SemaphoreType.DMA,
        pltpu.SemaphoreType.DMA,
        pltpu.SemaphoreType.DMA,
    ],
    compiler_params=pltpu.CompilerParams(needs_layout_passes=False),
)
def _interp_sc(t_hbm, us_hbm, out_hbm, us_v, t0_v, t1_v, o0_v, o1_v,
               us_sh, si0, si1, so0, so1):
    wid = lax.axis_index("s") * NC + lax.axis_index("c")
    base = wid * QPW

    def compute(t_ref, o_ref):
        @plsc.parallel_loop(0, CHUNK, 16, unroll=12)
        def _(i):
            tq = t_ref[pl.ds(i, 16)]
            c = tq * K
            i0 = c.astype(jnp.int32)
            idx = i0 + 1
            yi = plsc.load_gather(us_v, [idx])
            ym = plsc.load_gather(us_v, [i0])
            dy = yi - ym
            off = c - idx.astype(jnp.float32)
            o_ref[pl.ds(i, 16)] = yi + dy * off

    def fetch(g, t_ref, sem):
        pltpu.async_copy(t_hbm.at[pl.ds(base + g * CHUNK, CHUNK)], t_ref, sem)

    def wait_in(t_ref, sem):
        pltpu.make_async_copy(t_hbm.at[pl.ds(0, CHUNK)], t_ref, sem).wait()

    def put(g, o_ref, sem):
        pltpu.async_copy(o_ref, out_hbm.at[pl.ds(base + g * CHUNK, CHUNK)], sem)

    def wait_out(o_ref, sem):
        pltpu.make_async_copy(o_ref, out_hbm.at[pl.ds(0, CHUNK)], sem).wait()

    sid = lax.axis_index("s")
    fetch(0, t0_v, si0)
    fetch(1, t1_v, si1)

    @pl.when(sid == 0)
    def _():
        pltpu.sync_copy(us_hbm, us_sh)

    plsc.subcore_barrier()
    pltpu.sync_copy(us_sh, us_v)
    lane = jnp.arange(16, dtype=jnp.int32)
    us_v[pl.ds(0, 16)] = plsc.load_gather(us_v, [jnp.maximum(lane, 1)])
    wait_in(t0_v, si0)
    compute(t0_v, o0_v)
    put(0, o0_v, so0)
    fetch(2, t0_v, si0)
    wait_in(t1_v, si1)
    compute(t1_v, o1_v)
    put(1, o1_v, so1)
    fetch(3, t1_v, si1)

    def body(k, carry):
        g0 = 2 * k
        wait_in(t0_v, si0)
        wait_out(o0_v, so0)
        compute(t0_v, o0_v)

        @pl.when(g0 + 2 < NCHUNKS)
        def _():
            fetch(g0 + 2, t0_v, si0)

        put(g0, o0_v, so0)

        wait_in(t1_v, si1)
        wait_out(o1_v, so1)
        compute(t1_v, o1_v)

        @pl.when(g0 + 3 < NCHUNKS)
        def _():
            fetch(g0 + 3, t1_v, si1)

        put(g0 + 1, o1_v, so1)
        return carry

    lax.fori_loop(1, NCHUNKS // 2, body, 0)
    wait_out(o0_v, so0)
    wait_out(o1_v, so1)


def kernel(x, t, us, t_range):
    return _interp_sc(t, us)
